# baseline (speedup 1.0000x reference)
import functools

import jax
import jax.numpy as jnp
import numpy as np
from jax import lax
from jax.experimental import pallas as pl
from jax.experimental.pallas import tpu as pltpu
from jax.experimental.pallas import tpu_sc as plsc

_B, _V = 4096, 1000
_DISTANCE_WEIGHT = 1.0

_R = 512
_NW = 32
_BPW = _B // _NW
_L = 16


def _np_threefry2x32(k1, k2, x0, x1):
    def rotl(x, d):
        return (x << np.uint32(d)) | (x >> np.uint32(32 - d))

    rot = [(13, 15, 26, 6), (17, 29, 16, 24)]
    ks = [np.uint32(k1), np.uint32(k2),
          np.uint32(k1) ^ np.uint32(k2) ^ np.uint32(0x1BD11BDA)]
    x = [x0.astype(np.uint32) + ks[0], x1.astype(np.uint32) + ks[1]]
    for i in range(5):
        for r in rot[i % 2]:
            x[0] = x[0] + x[1]
            x[1] = rotl(x[1], r)
            x[1] = x[0] ^ x[1]
        x[0] = x[0] + ks[(i + 1) % 3]
        x[1] = x[1] + ks[(i + 2) % 3] + np.uint32(i + 1)
    return x


def _build_rank_const() -> np.ndarray:
    old = np.seterr(over="ignore")
    try:
        b1, b2 = _np_threefry2x32(0, 1, np.zeros(2, np.uint32),
                                  np.arange(2, dtype=np.uint32))
        subkeys = [(b1[0], b2[0]), (b1[1], b2[1])]

        def gumbel(k):
            n = _B * _V
            h1, h2 = _np_threefry2x32(k[0], k[1], np.zeros(n, np.uint32),
                                      np.arange(n, dtype=np.uint32))
            bits = h1 ^ h2
            fb = (bits >> np.uint32(9)) | np.uint32(0x3F800000)
            floats = fb.view(np.float32) - np.float32(1.0)
            tiny = np.float32(np.finfo(np.float32).tiny)
            u = np.maximum(tiny, floats * (np.float32(1.0) - tiny) + tiny)
            return -np.log(-np.log(u)).reshape(_B, _V)

        gp = gumbel(subkeys[0])
        gn = gumbel(subkeys[1])
    finally:
        np.seterr(**old)

    def ranks(g: np.ndarray) -> np.ndarray:
        order = np.argsort(-g, axis=1, kind="stable")
        r = np.empty((_B, _V), np.int32)
        r[np.arange(_B)[:, None], order] = np.arange(_V, dtype=np.int32)[None, :]
        return r

    return ((ranks(gp) << 10) | ranks(gn)).astype(np.int32)


_RANKC = _build_rank_const()


def _sample_body(t_ref, c_ref, x_ref, l_ref, d_ref):
    t = t_ref[...]
    c = c_ref[...]
    x = x_ref[...]
    j = lax.broadcasted_iota(jnp.int32, (_R, _V), 1)
    big = jnp.int32(1 << 30)
    packed_p = (c & 0xFFC00) | j
    packed_n = ((c & 0x3FF) << 10) | j
    mask = t == 1
    keyed_p = jnp.where(mask, packed_p, big)
    keyed_n = jnp.where(mask, big, packed_n)
    mp = jnp.min(keyed_p, axis=1)
    mn = jnp.min(keyed_n, axis=1)
    zero = jnp.float32(0.0)
    p = jnp.sum(jnp.where(keyed_p == mp[:, None], x, zero), axis=1)
    n = jnp.sum(jnp.where(keyed_n == mn[:, None], x, zero), axis=1)
    l1m = jnp.where(n < -0.6931471805599453,
                    jnp.log1p(-jnp.exp(n)),
                    jnp.log(-(jnp.exp(n) - 1.0)))
    l_ref[0, 0, :] = -p - l1m
    d_ref[0, 0, :] = (mp & 1023) * _V + (mn & 1023)


def _sample(target, rank_c, x):
    nb = _B // _R
    return pl.pallas_call(
        _sample_body,
        grid=(nb,),
        in_specs=[
            pl.BlockSpec((_R, _V), lambda i: (i, 0)),
            pl.BlockSpec((_R, _V), lambda i: (i, 0)),
            pl.BlockSpec((_R, _V), lambda i: (i, 0)),
        ],
        out_specs=[
            pl.BlockSpec((1, 1, _R), lambda i: (i, 0, 0)),
            pl.BlockSpec((1, 1, _R), lambda i: (i, 0, 0)),
        ],
        out_shape=[
            jax.ShapeDtypeStruct((nb, 1, _R), jnp.float32),
            jax.ShapeDtypeStruct((nb, 1, _R), jnp.int32),
        ],
    )(target, rank_c, x)


def _sc_gather(dm_flat, didx):
    mesh = plsc.VectorSubcoreMesh(core_axis_name="c", subcore_axis_name="s")

    @functools.partial(
        pl.kernel,
        mesh=mesh,
        out_type=jax.ShapeDtypeStruct((_B,), jnp.float32),
        scratch_types=[
            pltpu.VMEM((_BPW,), jnp.int32),
            pltpu.VMEM((_BPW,), jnp.float32),
            pltpu.SemaphoreType.DMA,
        ],
    )
    def k(dm_hbm, didx_hbm, dvec_out, didx_v, dval, sem):
        wid = lax.axis_index("s") * 2 + lax.axis_index("c")
        base = wid * _BPW
        pltpu.sync_copy(didx_hbm.at[pl.ds(base, _BPW)], didx_v)
        pltpu.async_copy(dm_hbm.at[didx_v], dval, sem).wait()
        pltpu.sync_copy(dval, dvec_out.at[pl.ds(base, _BPW)])

    return k(dm_flat, didx)


def _finish_body(l_ref, d_ref, o_ref):
    w = l_ref[...] * (d_ref[...] * _DISTANCE_WEIGHT)
    o_ref[0, 0] = jnp.sum(w) / _B


def _finish(l_part, dvec):
    rows, cols = 8, _B // 8
    return pl.pallas_call(
        _finish_body,
        in_specs=[pl.BlockSpec((rows, cols), lambda: (0, 0))] * 2,
        out_specs=pl.BlockSpec(memory_space=pltpu.SMEM),
        out_shape=jax.ShapeDtypeStruct((1, 1), jnp.float32),
    )(l_part.reshape(rows, cols), dvec.reshape(rows, cols))


def kernel(input, target, distance_matrix):
    rank_c = jnp.asarray(_RANKC)
    l2, d2 = _sample(target, rank_c, input)
    l_part = l2.reshape(_B)
    didx = d2.reshape(_B)
    dvec = _sc_gather(distance_matrix.reshape(_V * _V), didx)
    out = _finish(l_part, dvec)
    return out.reshape(())


# speedup vs baseline: 1.6952x; 1.6952x over previous
import functools

import jax
import jax.numpy as jnp
import numpy as np
from jax import lax
from jax.experimental import pallas as pl
from jax.experimental.pallas import tpu as pltpu
from jax.experimental.pallas import tpu_sc as plsc

_B, _V = 4096, 1000
_DISTANCE_WEIGHT = 1.0

_R = 512
_NW = 32
_BPW = _B // _NW
_L = 16


def _np_threefry2x32(k1, k2, x0, x1):
    def rotl(x, d):
        return (x << np.uint32(d)) | (x >> np.uint32(32 - d))

    rot = [(13, 15, 26, 6), (17, 29, 16, 24)]
    ks = [np.uint32(k1), np.uint32(k2),
          np.uint32(k1) ^ np.uint32(k2) ^ np.uint32(0x1BD11BDA)]
    x = [x0.astype(np.uint32) + ks[0], x1.astype(np.uint32) + ks[1]]
    for i in range(5):
        for r in rot[i % 2]:
            x[0] = x[0] + x[1]
            x[1] = rotl(x[1], r)
            x[1] = x[0] ^ x[1]
        x[0] = x[0] + ks[(i + 1) % 3]
        x[1] = x[1] + ks[(i + 2) % 3] + np.uint32(i + 1)
    return x


def _build_rank_const() -> np.ndarray:
    old = np.seterr(over="ignore")
    try:
        b1, b2 = _np_threefry2x32(0, 1, np.zeros(2, np.uint32),
                                  np.arange(2, dtype=np.uint32))
        subkeys = [(b1[0], b2[0]), (b1[1], b2[1])]

        def gumbel(k):
            n = _B * _V
            h1, h2 = _np_threefry2x32(k[0], k[1], np.zeros(n, np.uint32),
                                      np.arange(n, dtype=np.uint32))
            bits = h1 ^ h2
            fb = (bits >> np.uint32(9)) | np.uint32(0x3F800000)
            floats = fb.view(np.float32) - np.float32(1.0)
            tiny = np.float32(np.finfo(np.float32).tiny)
            u = np.maximum(tiny, floats * (np.float32(1.0) - tiny) + tiny)
            return -np.log(-np.log(u)).reshape(_B, _V)

        gp = gumbel(subkeys[0])
        gn = gumbel(subkeys[1])
    finally:
        np.seterr(**old)

    def ranks(g: np.ndarray) -> np.ndarray:
        order = np.argsort(-g, axis=1, kind="stable")
        r = np.empty((_B, _V), np.int32)
        r[np.arange(_B)[:, None], order] = np.arange(_V, dtype=np.int32)[None, :]
        return r

    return ((ranks(gp) << 10) | ranks(gn)).astype(np.int32)


_RANKC_T = np.ascontiguousarray(_build_rank_const().T)


def _sample_body(t_ref, c_ref, x_ref, l_ref, d_ref):
    t = t_ref[...]
    c = c_ref[...]
    x = x_ref[...]
    j = lax.broadcasted_iota(jnp.int32, (_V, _R), 0)
    big = jnp.int32(1 << 30)
    packed_p = (c & 0xFFC00) | j
    packed_n = ((c & 0x3FF) << 10) | j
    mask = t == 1
    keyed_p = jnp.where(mask, packed_p, big)
    keyed_n = jnp.where(mask, big, packed_n)
    mp = jnp.min(keyed_p, axis=0)
    mn = jnp.min(keyed_n, axis=0)
    zero = jnp.float32(0.0)
    p = jnp.sum(jnp.where(keyed_p == mp[None, :], x, zero), axis=0)
    n = jnp.sum(jnp.where(keyed_n == mn[None, :], x, zero), axis=0)
    l1m = jnp.where(n < -0.6931471805599453,
                    jnp.log1p(-jnp.exp(n)),
                    jnp.log(-(jnp.exp(n) - 1.0)))
    l_ref[0, 0, :] = -p - l1m
    d_ref[0, 0, :] = (mp & 1023) * _V + (mn & 1023)


def _sample(target_t, rank_c_t, x_t):
    nb = _B // _R
    return pl.pallas_call(
        _sample_body,
        grid=(nb,),
        in_specs=[
            pl.BlockSpec((_V, _R), lambda i: (0, i)),
            pl.BlockSpec((_V, _R), lambda i: (0, i)),
            pl.BlockSpec((_V, _R), lambda i: (0, i)),
        ],
        out_specs=[
            pl.BlockSpec((1, 1, _R), lambda i: (i, 0, 0)),
            pl.BlockSpec((1, 1, _R), lambda i: (i, 0, 0)),
        ],
        out_shape=[
            jax.ShapeDtypeStruct((nb, 1, _R), jnp.float32),
            jax.ShapeDtypeStruct((nb, 1, _R), jnp.int32),
        ],
    )(target_t, rank_c_t, x_t)


def _sc_gather(dm_flat, didx):
    mesh = plsc.VectorSubcoreMesh(core_axis_name="c", subcore_axis_name="s")

    @functools.partial(
        pl.kernel,
        mesh=mesh,
        out_type=jax.ShapeDtypeStruct((_B,), jnp.float32),
        scratch_types=[
            pltpu.VMEM((_BPW,), jnp.int32),
            pltpu.VMEM((_BPW,), jnp.float32),
            pltpu.SemaphoreType.DMA,
        ],
    )
    def k(dm_hbm, didx_hbm, dvec_out, didx_v, dval, sem):
        wid = lax.axis_index("s") * 2 + lax.axis_index("c")
        base = wid * _BPW
        pltpu.sync_copy(didx_hbm.at[pl.ds(base, _BPW)], didx_v)
        pltpu.async_copy(dm_hbm.at[didx_v], dval, sem).wait()
        pltpu.sync_copy(dval, dvec_out.at[pl.ds(base, _BPW)])

    return k(dm_flat, didx)


def _finish_body(l_ref, d_ref, o_ref):
    w = l_ref[...] * (d_ref[...] * _DISTANCE_WEIGHT)
    o_ref[0, 0] = jnp.sum(w) / _B


def _finish(l_part, dvec):
    rows, cols = 8, _B // 8
    return pl.pallas_call(
        _finish_body,
        in_specs=[pl.BlockSpec((rows, cols), lambda: (0, 0))] * 2,
        out_specs=pl.BlockSpec(memory_space=pltpu.SMEM),
        out_shape=jax.ShapeDtypeStruct((1, 1), jnp.float32),
    )(l_part.reshape(rows, cols), dvec.reshape(rows, cols))


def kernel(input, target, distance_matrix):
    rank_c_t = jnp.asarray(_RANKC_T)
    l2, d2 = _sample(target.T, rank_c_t, input.T)
    l_part = l2.reshape(_B)
    didx = d2.reshape(_B)
    dvec = _sc_gather(distance_matrix.reshape(_V * _V), didx)
    out = _finish(l_part, dvec)
    return out.reshape(())


# speedup vs baseline: 1.6970x; 1.0010x over previous
import functools

import jax
import jax.numpy as jnp
import numpy as np
from jax import lax
from jax.experimental import pallas as pl
from jax.experimental.pallas import tpu as pltpu
from jax.experimental.pallas import tpu_sc as plsc

_B, _V = 4096, 1000
_DISTANCE_WEIGHT = 1.0

_R = 512
_NW = 32
_BPW = _B // _NW
_L = 16


def _np_threefry2x32(k1, k2, x0, x1):
    def rotl(x, d):
        return (x << np.uint32(d)) | (x >> np.uint32(32 - d))

    rot = [(13, 15, 26, 6), (17, 29, 16, 24)]
    ks = [np.uint32(k1), np.uint32(k2),
          np.uint32(k1) ^ np.uint32(k2) ^ np.uint32(0x1BD11BDA)]
    x = [x0.astype(np.uint32) + ks[0], x1.astype(np.uint32) + ks[1]]
    for i in range(5):
        for r in rot[i % 2]:
            x[0] = x[0] + x[1]
            x[1] = rotl(x[1], r)
            x[1] = x[0] ^ x[1]
        x[0] = x[0] + ks[(i + 1) % 3]
        x[1] = x[1] + ks[(i + 2) % 3] + np.uint32(i + 1)
    return x


def _build_rank_const() -> np.ndarray:
    old = np.seterr(over="ignore")
    try:
        b1, b2 = _np_threefry2x32(0, 1, np.zeros(2, np.uint32),
                                  np.arange(2, dtype=np.uint32))
        subkeys = [(b1[0], b2[0]), (b1[1], b2[1])]

        def gumbel(k):
            n = _B * _V
            h1, h2 = _np_threefry2x32(k[0], k[1], np.zeros(n, np.uint32),
                                      np.arange(n, dtype=np.uint32))
            bits = h1 ^ h2
            fb = (bits >> np.uint32(9)) | np.uint32(0x3F800000)
            floats = fb.view(np.float32) - np.float32(1.0)
            tiny = np.float32(np.finfo(np.float32).tiny)
            u = np.maximum(tiny, floats * (np.float32(1.0) - tiny) + tiny)
            return -np.log(-np.log(u)).reshape(_B, _V)

        gp = gumbel(subkeys[0])
        gn = gumbel(subkeys[1])
    finally:
        np.seterr(**old)

    def ranks(g: np.ndarray) -> np.ndarray:
        order = np.argsort(-g, axis=1, kind="stable")
        r = np.empty((_B, _V), np.int32)
        r[np.arange(_B)[:, None], order] = np.arange(_V, dtype=np.int32)[None, :]
        return r

    return ((ranks(gp) << 10) | ranks(gn)).astype(np.int32)


_RANKC_T = np.ascontiguousarray(_build_rank_const().T)


def _sample_body(t_ref, c_ref, x_ref, l_ref, d_ref):
    t = t_ref[...]
    c = c_ref[...]
    x = x_ref[...]
    j = lax.broadcasted_iota(jnp.int32, (_V, _R), 0)
    big = jnp.int32(1 << 30)
    packed_p = (c & 0xFFC00) | j
    packed_n = ((c & 0x3FF) << 10) | j
    mask = t == 1
    keyed_p = jnp.where(mask, packed_p, big)
    keyed_n = jnp.where(mask, big, packed_n)
    mp = jnp.min(keyed_p, axis=0)
    mn = jnp.min(keyed_n, axis=0)
    zero = jnp.float32(0.0)
    p = jnp.sum(jnp.where(keyed_p == mp[None, :], x, zero), axis=0)
    n = jnp.sum(jnp.where(keyed_n == mn[None, :], x, zero), axis=0)
    l1m = jnp.where(n < -0.6931471805599453,
                    jnp.log1p(-jnp.exp(n)),
                    jnp.log(-(jnp.exp(n) - 1.0)))
    l_ref[...] = -p - l1m
    d_ref[...] = (mp & 1023) * _V + (mn & 1023)


def _sample(target_t, rank_c_t, x_t):
    nb = _B // _R
    return pl.pallas_call(
        _sample_body,
        grid=(nb,),
        in_specs=[
            pl.BlockSpec((_V, _R), lambda i: (0, i)),
            pl.BlockSpec((_V, _R), lambda i: (0, i)),
            pl.BlockSpec((_V, _R), lambda i: (0, i)),
        ],
        out_specs=[
            pl.BlockSpec((_R,), lambda i: (i,)),
            pl.BlockSpec((_R,), lambda i: (i,)),
        ],
        out_shape=[
            jax.ShapeDtypeStruct((_B,), jnp.float32),
            jax.ShapeDtypeStruct((_B,), jnp.int32),
        ],
    )(target_t, rank_c_t, x_t)


def _sc_gather(dm_flat, didx):
    mesh = plsc.VectorSubcoreMesh(core_axis_name="c", subcore_axis_name="s")

    @functools.partial(
        pl.kernel,
        mesh=mesh,
        out_type=jax.ShapeDtypeStruct((_B,), jnp.float32),
        scratch_types=[
            pltpu.VMEM((_BPW,), jnp.int32),
            pltpu.VMEM((_BPW,), jnp.float32),
            pltpu.SemaphoreType.DMA,
        ],
    )
    def k(dm_hbm, didx_hbm, dvec_out, didx_v, dval, sem):
        wid = lax.axis_index("s") * 2 + lax.axis_index("c")
        base = wid * _BPW
        pltpu.sync_copy(didx_hbm.at[pl.ds(base, _BPW)], didx_v)
        pltpu.async_copy(dm_hbm.at[didx_v], dval, sem).wait()
        pltpu.sync_copy(dval, dvec_out.at[pl.ds(base, _BPW)])

    return k(dm_flat, didx)


def _finish_body(l_ref, d_ref, o_ref):
    w = l_ref[...] * (d_ref[...] * _DISTANCE_WEIGHT)
    o_ref[0, 0] = jnp.sum(w) / _B


def _finish(l_part, dvec):
    rows, cols = 8, _B // 8
    return pl.pallas_call(
        _finish_body,
        in_specs=[pl.BlockSpec((rows, cols), lambda: (0, 0))] * 2,
        out_specs=pl.BlockSpec(memory_space=pltpu.SMEM),
        out_shape=jax.ShapeDtypeStruct((1, 1), jnp.float32),
    )(l_part.reshape(rows, cols), dvec.reshape(rows, cols))


def kernel(input, target, distance_matrix):
    rank_c_t = jnp.asarray(_RANKC_T)
    dm_flat = distance_matrix.reshape(_V * _V)
    l_part, didx = _sample(target.T, rank_c_t, input.T)
    dvec = _sc_gather(dm_flat, didx)
    out = _finish(l_part, dvec)
    return out.reshape(())
